# Initial kernel scaffold; baseline (speedup 1.0000x reference)
import jax
import jax.numpy as jnp
from jax.experimental import pallas as pl


def kernel(x, wte, wpe, gamma, beta, W, b):
    raise NotImplementedError("write your pallas kernel here")


# speedup vs baseline: 1.3266x; 1.3266x over previous
import functools

import jax
import jax.numpy as jnp
from jax import lax
from jax.experimental import pallas as pl
from jax.experimental.pallas import tpu as pltpu
from jax.experimental.pallas import tpu_sc as plsc

EPS = 1e-5


@functools.partial(jax.jit, static_argnums=(2, 3))
def _sc_gather(wte, idx, ntok, d):
    NC, NS = 2, 16
    NW = NC * NS
    b_per_w = ntok // NW
    CH = 64
    nchunk = b_per_w // CH

    mesh = plsc.VectorSubcoreMesh(core_axis_name="c", subcore_axis_name="s")

    @functools.partial(
        pl.kernel,
        mesh=mesh,
        out_type=jax.ShapeDtypeStruct((ntok, d), jnp.float32),
        scratch_types=[
            pltpu.VMEM((CH,), jnp.int32),
            pltpu.VMEM((CH, d), jnp.float32),
            pltpu.SemaphoreType.DMA,
        ],
    )
    def gather_kernel(table_hbm, idx_hbm, out_hbm, idx_v, rows_v, sem):
        wid = lax.axis_index("s") * NC + lax.axis_index("c")
        base = wid * b_per_w
        for c in range(nchunk):
            off = base + c * CH
            pltpu.sync_copy(idx_hbm.at[pl.ds(off, CH)], idx_v)
            pltpu.async_copy(table_hbm.at[idx_v], rows_v, sem).wait()
            pltpu.sync_copy(rows_v, out_hbm.at[pl.ds(off, CH)])

    return gather_kernel(wte, idx)


def _tc_body(tok_ref, wpe_ref, gamma_ref, beta_ref, w_ref, b_ref, out_ref):
    y = tok_ref[...] + wpe_ref[...]
    mu = jnp.mean(y, axis=1, keepdims=True)
    yc = y - mu
    var = jnp.mean(yc * yc, axis=1, keepdims=True)
    z = yc * lax.rsqrt(var + EPS) * gamma_ref[...] + beta_ref[...]
    out_ref[...] = (
        lax.dot_general(z, w_ref[...], (((1,), (1,)), ((), ())),
                        preferred_element_type=jnp.float32)
        + b_ref[...]
    )


@functools.partial(jax.jit, static_argnums=(6,))
def _tc_ln_proj(tokens, wpe, gamma, beta, W, b, t_period):
    ntok, d = tokens.shape
    BLK = 512
    nblk = ntok // BLK
    wpe_blocks = t_period // BLK

    return pl.pallas_call(
        _tc_body,
        grid=(nblk,),
        in_specs=[
            pl.BlockSpec((BLK, d), lambda i: (i, 0)),
            pl.BlockSpec((BLK, d), lambda i: (i % wpe_blocks, 0)),
            pl.BlockSpec((1, d), lambda i: (0, 0)),
            pl.BlockSpec((1, d), lambda i: (0, 0)),
            pl.BlockSpec((d, d), lambda i: (0, 0)),
            pl.BlockSpec((1, d), lambda i: (0, 0)),
        ],
        out_specs=pl.BlockSpec((BLK, d), lambda i: (i, 0)),
        out_shape=jax.ShapeDtypeStruct((ntok, d), jnp.float32),
    )(tokens, wpe, gamma, beta, W, b)


def kernel(x, wte, wpe, gamma, beta, W, b):
    B, T = x.shape
    V, D = wte.shape
    idx = x.reshape(-1).astype(jnp.int32)
    tokens = _sc_gather(wte, idx, B * T, D)
    out = _tc_ln_proj(tokens, wpe, gamma.reshape(1, D), beta.reshape(1, D),
                      W, b.reshape(1, D), T)
    return out.reshape(B, T, D)


# speedup vs baseline: 1.3293x; 1.0020x over previous
import functools

import jax
import jax.numpy as jnp
from jax import lax
from jax.experimental import pallas as pl
from jax.experimental.pallas import tpu as pltpu
from jax.experimental.pallas import tpu_sc as plsc

EPS = 1e-5


@functools.partial(jax.jit, static_argnums=(2, 3))
def _sc_gather(wte, idx, ntok, d):
    NC, NS = 2, 16
    NW = NC * NS
    b_per_w = ntok // NW
    CH = 64
    nchunk = b_per_w // CH

    mesh = plsc.VectorSubcoreMesh(core_axis_name="c", subcore_axis_name="s")

    @functools.partial(
        pl.kernel,
        mesh=mesh,
        out_type=jax.ShapeDtypeStruct((ntok, d), jnp.float32),
        scratch_types=[
            pltpu.VMEM((CH,), jnp.int32),
            pltpu.VMEM((CH, d), jnp.float32),
            pltpu.SemaphoreType.DMA,
        ],
    )
    def gather_kernel(table_hbm, idx_hbm, out_hbm, idx_v, rows_v, sem):
        wid = lax.axis_index("s") * NC + lax.axis_index("c")
        base = wid * b_per_w
        for c in range(nchunk):
            off = base + c * CH
            pltpu.sync_copy(idx_hbm.at[pl.ds(off, CH)], idx_v)
            pltpu.async_copy(table_hbm.at[idx_v], rows_v, sem).wait()
            pltpu.sync_copy(rows_v, out_hbm.at[pl.ds(off, CH)])

    return gather_kernel(wte, idx)


def _tc_body(tok_ref, wpe_ref, gamma_ref, beta_ref, w_ref, b_ref, out_ref):
    y = tok_ref[...] + wpe_ref[...]
    mu = jnp.mean(y, axis=1, keepdims=True)
    yc = y - mu
    var = jnp.mean(yc * yc, axis=1, keepdims=True)
    z = yc * lax.rsqrt(var + EPS) * gamma_ref[...] + beta_ref[...]
    out_ref[...] = (
        lax.dot_general(z.astype(jnp.bfloat16), w_ref[...],
                        (((1,), (1,)), ((), ())),
                        preferred_element_type=jnp.float32)
        + b_ref[...]
    )


@functools.partial(jax.jit, static_argnums=(6,))
def _tc_ln_proj(tokens, wpe, gamma, beta, W, b, t_period):
    ntok, d = tokens.shape
    BLK = 512
    nblk = ntok // BLK
    wpe_blocks = t_period // BLK

    return pl.pallas_call(
        _tc_body,
        grid=(nblk,),
        in_specs=[
            pl.BlockSpec((BLK, d), lambda i: (i, 0)),
            pl.BlockSpec((BLK, d), lambda i: (i % wpe_blocks, 0)),
            pl.BlockSpec((1, d), lambda i: (0, 0)),
            pl.BlockSpec((1, d), lambda i: (0, 0)),
            pl.BlockSpec((d, d), lambda i: (0, 0)),
            pl.BlockSpec((1, d), lambda i: (0, 0)),
        ],
        out_specs=pl.BlockSpec((BLK, d), lambda i: (i, 0)),
        out_shape=jax.ShapeDtypeStruct((ntok, d), jnp.float32),
    )(tokens, wpe, gamma, beta, W, b)


def kernel(x, wte, wpe, gamma, beta, W, b):
    B, T = x.shape
    V, D = wte.shape
    idx = x.reshape(-1).astype(jnp.int32)
    tokens = _sc_gather(wte, idx, B * T, D)
    out = _tc_ln_proj(tokens, wpe, gamma.reshape(1, D), beta.reshape(1, D),
                      W.astype(jnp.bfloat16), b.reshape(1, D), T)
    return out.reshape(B, T, D)


# speedup vs baseline: 1.3746x; 1.0341x over previous
import functools

import jax
import jax.numpy as jnp
from jax import lax
from jax.experimental import pallas as pl
from jax.experimental.pallas import tpu as pltpu
from jax.experimental.pallas import tpu_sc as plsc

EPS = 1e-5


@functools.partial(jax.jit, static_argnums=(2, 3))
def _sc_gather(wte, idx, ntok, d):
    NC, NS = 2, 16
    NW = NC * NS
    b_per_w = ntok // NW
    CH = 64
    nchunk = b_per_w // CH

    mesh = plsc.VectorSubcoreMesh(core_axis_name="c", subcore_axis_name="s")

    @functools.partial(
        pl.kernel,
        mesh=mesh,
        out_type=jax.ShapeDtypeStruct((ntok, d), jnp.float32),
        scratch_types=[
            pltpu.VMEM((b_per_w,), jnp.int32),
            pltpu.VMEM((CH, d), jnp.float32),
            pltpu.VMEM((CH, d), jnp.float32),
            pltpu.SemaphoreType.DMA,
            pltpu.SemaphoreType.DMA,
            pltpu.SemaphoreType.DMA,
            pltpu.SemaphoreType.DMA,
        ],
    )
    def gather_kernel(table_hbm, idx_hbm, out_hbm, idx_v, rows0, rows1,
                      gsem0, gsem1, wsem0, wsem1):
        wid = lax.axis_index("s") * NC + lax.axis_index("c")
        base = wid * b_per_w
        rows = (rows0, rows1)
        gsems = (gsem0, gsem1)
        wsems = (wsem0, wsem1)
        pltpu.sync_copy(idx_hbm.at[pl.ds(base, b_per_w)], idx_v)

        def gather_start(c):
            return pltpu.async_copy(
                table_hbm.at[idx_v.at[pl.ds(c * CH, CH)]],
                rows[c % 2], gsems[c % 2])

        gcopies = [None] * nchunk
        wcopies = [None] * nchunk
        gcopies[0] = gather_start(0)
        for c in range(nchunk):
            if c + 1 < nchunk:
                if c >= 1:
                    wcopies[c - 1].wait()
                gcopies[c + 1] = gather_start(c + 1)
            gcopies[c].wait()
            wcopies[c] = pltpu.async_copy(
                rows[c % 2], out_hbm.at[pl.ds(base + c * CH, CH)],
                wsems[c % 2])
        wcopies[nchunk - 2].wait()
        wcopies[nchunk - 1].wait()

    return gather_kernel(wte, idx)


def _tc_body(tok_ref, wpe_ref, gamma_ref, beta_ref, w_ref, b_ref, out_ref):
    y = tok_ref[...] + wpe_ref[...]
    mu = jnp.mean(y, axis=1, keepdims=True)
    yc = y - mu
    var = jnp.mean(yc * yc, axis=1, keepdims=True)
    z = yc * lax.rsqrt(var + EPS) * gamma_ref[...] + beta_ref[...]
    out_ref[...] = (
        lax.dot_general(z.astype(jnp.bfloat16), w_ref[...],
                        (((1,), (1,)), ((), ())),
                        preferred_element_type=jnp.float32)
        + b_ref[...]
    )


@functools.partial(jax.jit, static_argnums=(6,))
def _tc_ln_proj(tokens, wpe, gamma, beta, W, b, t_period):
    ntok, d = tokens.shape
    BLK = 512
    nblk = ntok // BLK
    wpe_blocks = t_period // BLK

    return pl.pallas_call(
        _tc_body,
        grid=(nblk,),
        in_specs=[
            pl.BlockSpec((BLK, d), lambda i: (i, 0)),
            pl.BlockSpec((BLK, d), lambda i: (i % wpe_blocks, 0)),
            pl.BlockSpec((1, d), lambda i: (0, 0)),
            pl.BlockSpec((1, d), lambda i: (0, 0)),
            pl.BlockSpec((d, d), lambda i: (0, 0)),
            pl.BlockSpec((1, d), lambda i: (0, 0)),
        ],
        out_specs=pl.BlockSpec((BLK, d), lambda i: (i, 0)),
        out_shape=jax.ShapeDtypeStruct((ntok, d), jnp.float32),
    )(tokens, wpe, gamma, beta, W, b)


def kernel(x, wte, wpe, gamma, beta, W, b):
    B, T = x.shape
    V, D = wte.shape
    idx = x.reshape(-1).astype(jnp.int32)
    tokens = _sc_gather(wte, idx, B * T, D)
    out = _tc_ln_proj(tokens, wpe, gamma.reshape(1, D), beta.reshape(1, D),
                      W.astype(jnp.bfloat16), b.reshape(1, D), T)
    return out.reshape(B, T, D)


# speedup vs baseline: 1.5369x; 1.1181x over previous
import functools

import jax
import jax.numpy as jnp
from jax import lax
from jax.experimental import pallas as pl
from jax.experimental.pallas import tpu as pltpu
from jax.experimental.pallas import tpu_sc as plsc

EPS = 1e-5


@functools.partial(jax.jit, static_argnums=(2, 3))
def _sc_gather(wte, idx, ntok, d):
    NC, NS = 2, 16
    NW = NC * NS
    b_per_w = ntok // NW
    CH = 64
    nchunk = b_per_w // CH

    mesh = plsc.VectorSubcoreMesh(core_axis_name="c", subcore_axis_name="s")

    @functools.partial(
        pl.kernel,
        mesh=mesh,
        out_type=jax.ShapeDtypeStruct((ntok, d), jnp.float32),
        scratch_types=[
            pltpu.VMEM((b_per_w,), jnp.int32),
            pltpu.VMEM((CH, d), jnp.float32),
            pltpu.VMEM((CH, d), jnp.float32),
            pltpu.SemaphoreType.DMA,
            pltpu.SemaphoreType.DMA,
            pltpu.SemaphoreType.DMA,
            pltpu.SemaphoreType.DMA,
        ],
    )
    def gather_kernel(table_hbm, idx_hbm, out_hbm, idx_v, rows0, rows1,
                      gsem0, gsem1, wsem0, wsem1):
        wid = lax.axis_index("s") * NC + lax.axis_index("c")
        base = wid * b_per_w
        rows = (rows0, rows1)
        gsems = (gsem0, gsem1)
        wsems = (wsem0, wsem1)
        pltpu.sync_copy(idx_hbm.at[pl.ds(base, b_per_w)], idx_v)

        def gather_start(c):
            return pltpu.async_copy(
                table_hbm.at[idx_v.at[pl.ds(c * CH, CH)]],
                rows[c % 2], gsems[c % 2])

        gcopies = [None] * nchunk
        wcopies = [None] * nchunk
        gcopies[0] = gather_start(0)
        for c in range(nchunk):
            if c + 1 < nchunk:
                if c >= 1:
                    wcopies[c - 1].wait()
                gcopies[c + 1] = gather_start(c + 1)
            gcopies[c].wait()
            wcopies[c] = pltpu.async_copy(
                rows[c % 2], out_hbm.at[pl.ds(base + c * CH, CH)],
                wsems[c % 2])
        wcopies[nchunk - 2].wait()
        wcopies[nchunk - 1].wait()

    return gather_kernel(wte, idx)


def _tc_body(tok_ref, wpe_ref, gamma_ref, beta_ref, w_ref, b_ref, out_ref):
    y = tok_ref[...] + wpe_ref[...]
    mu = jnp.mean(y, axis=1, keepdims=True)
    yc = y - mu
    var = jnp.mean(yc * yc, axis=1, keepdims=True)
    z = yc * lax.rsqrt(var + EPS) * gamma_ref[...] + beta_ref[...]
    out_ref[...] = (
        lax.dot_general(z, w_ref[...], (((1,), (1,)), ((), ())),
                        preferred_element_type=jnp.float32)
        + b_ref[...]
    )


@functools.partial(jax.jit, static_argnums=(6,))
def _tc_ln_proj(tokens, wpe, gamma, beta, W, b, t_period):
    ntok, d = tokens.shape
    BLK = t_period
    nblk = ntok // BLK

    return pl.pallas_call(
        _tc_body,
        grid=(nblk,),
        in_specs=[
            pl.BlockSpec((BLK, d), lambda i: (i, 0)),
            pl.BlockSpec((BLK, d), lambda i: (0, 0)),
            pl.BlockSpec((1, d), lambda i: (0, 0)),
            pl.BlockSpec((1, d), lambda i: (0, 0)),
            pl.BlockSpec((d, d), lambda i: (0, 0)),
            pl.BlockSpec((1, d), lambda i: (0, 0)),
        ],
        out_specs=pl.BlockSpec((BLK, d), lambda i: (i, 0)),
        out_shape=jax.ShapeDtypeStruct((ntok, d), jnp.float32),
    )(tokens, wpe, gamma, beta, W, b)


def kernel(x, wte, wpe, gamma, beta, W, b):
    B, T = x.shape
    V, D = wte.shape
    idx = x.reshape(-1).astype(jnp.int32)
    tokens = _sc_gather(wte, idx, B * T, D)
    out = _tc_ln_proj(tokens, wpe, gamma.reshape(1, D), beta.reshape(1, D),
                      W, b.reshape(1, D), T)
    return out.reshape(B, T, D)
